# baseline (speedup 1.0000x reference)
import functools

import jax
import jax.numpy as jnp
from jax.experimental import pallas as pl
from jax.experimental.pallas import tpu as pltpu

C_PAD = 128
NC = 4


def _table_kernel(tok_ref, b1_ref, w2p_ref, p_ref, emb_hbm, w1_hbm, out_ref,
                  emb_vmem, w1_vmem, sems, *, bs, ve, vc, e, h):
    i = pl.program_id(0)

    pltpu.make_async_copy(w1_hbm, w1_vmem, sems.at[NC]).start()
    for c in range(NC):
        pltpu.make_async_copy(
            emb_hbm.at[pl.ds(i * ve + c * vc, vc), :],
            emb_vmem.at[pl.ds(c * vc, vc), :],
            sems.at[c]).start()

    iota = jax.lax.broadcasted_iota(jnp.int32, (bs, ve), 1) + i * ve
    oh = (tok_ref[...] == iota).astype(jnp.bfloat16)
    counts = jnp.dot(p_ref[...], oh,
                     preferred_element_type=jnp.float32)

    pltpu.make_async_copy(w1_vmem, w1_vmem, sems.at[NC]).wait()
    w1bf = w1_vmem[...].astype(jnp.bfloat16)

    acc = jnp.zeros(out_ref.shape[1:], jnp.float32)
    for c in range(NC):
        pltpu.make_async_copy(emb_vmem.at[pl.ds(c * vc, vc), :],
                              emb_vmem.at[pl.ds(c * vc, vc), :],
                              sems.at[c]).wait()
        embc = emb_vmem[c * vc:(c + 1) * vc, :].astype(jnp.bfloat16)
        wf = jnp.dot(embc, w1bf, preferred_element_type=jnp.float32)
        t = jnp.tanh(wf + b1_ref[...])
        u = jnp.dot(t, w2p_ref[...],
                    preferred_element_type=jnp.float32)
        acc = acc + jnp.dot(counts[:, c * vc:(c + 1) * vc], u,
                            preferred_element_type=jnp.float32)
    out_ref[0] = acc


@jax.jit
def kernel(tokens, emb, w1, b1, w2, b2):
    B, S = tokens.shape
    V, E = emb.shape
    H = w1.shape[1]
    C = w2.shape[1]
    VE = V // 2
    VC = VE // NC
    BS = B * S

    w2p = jnp.zeros((H, C_PAD), jnp.float32).at[:, :C].set(w2) * (1.0 / S)
    row_of = jnp.repeat(jnp.arange(B, dtype=jnp.int32), S)
    p_sel = (jnp.arange(B, dtype=jnp.int32)[:, None] == row_of[None, :]
             ).astype(jnp.bfloat16)
    tok_flat = tokens.reshape(BS, 1).astype(jnp.int32)

    flops = 2 * V * E * H + 2 * B * BS * V + 2 * B * V * C_PAD
    cost = pl.CostEstimate(flops=flops, transcendentals=V * H,
                           bytes_accessed=4 * (V * E + E * H + V * H))

    parts = pl.pallas_call(
        functools.partial(_table_kernel, bs=BS, ve=VE, vc=VC, e=E, h=H),
        out_shape=jax.ShapeDtypeStruct((2, B, C_PAD), jnp.float32),
        grid=(2,),
        in_specs=[
            pl.BlockSpec((BS, 1), lambda i: (0, 0)),
            pl.BlockSpec((1, H), lambda i: (0, 0)),
            pl.BlockSpec((H, C_PAD), lambda i: (0, 0)),
            pl.BlockSpec((B, BS), lambda i: (0, 0)),
            pl.BlockSpec(memory_space=pl.ANY),
            pl.BlockSpec(memory_space=pl.ANY),
        ],
        out_specs=pl.BlockSpec((1, B, C_PAD), lambda i: (i, 0, 0)),
        scratch_shapes=[
            pltpu.VMEM((VE, E), jnp.float32),
            pltpu.VMEM((E, H), jnp.float32),
            pltpu.SemaphoreType.DMA((NC + 1,)),
        ],
        compiler_params=pltpu.CompilerParams(
            dimension_semantics=("parallel",)),
        cost_estimate=cost,
    )(tok_flat, b1, w2p, p_sel, emb, w1)

    return parts.sum(axis=0)[:, :C] + b2
